# baseline (speedup 1.0000x reference)
import functools

import jax
import jax.numpy as jnp
from jax import lax
from jax.experimental import pallas as pl
from jax.experimental.pallas import tpu as pltpu
from jax.experimental.pallas import tpu_sc as plsc

N = 10000
NPAD = 10240
C = 128
NC, NS = 2, 16
NT = NC * NS
RPS = NPAD // NS
BLK = 2048
TRASH = N
NBUF = 4

_HI = lax.Precision.HIGHEST
_SC_PARAMS = pltpu.CompilerParams(use_tc_tiling_on_sc=False)
_MESH = plsc.VectorSubcoreMesh(core_axis_name="c", subcore_axis_name="s")


def _row_mask(i):
    rows = lax.broadcasted_iota(jnp.int32, (BLK, 1), 0) + i * BLK
    return rows < N


def _sc_scatter(ys, src, dst):
    d = ys[0].shape[1]
    assert all(y.shape[1] == d for y in ys)
    nbuf = 4 if d >= 64 else 8
    cc = src.shape[2]
    nchunk = src.shape[1]
    assert nchunk % nbuf == 0

    @functools.partial(
        pl.kernel,
        mesh=_MESH,
        out_type=[jax.ShapeDtypeStruct((NC, NPAD, d), jnp.float32)
                  for _ in ys],
        scratch_types=[
            pltpu.VMEM((nchunk, cc), jnp.int32),
            pltpu.VMEM((nchunk, cc), jnp.int32),
            *[pltpu.VMEM((cc, d), jnp.float32) for _ in range(nbuf)],
            pltpu.VMEM_SHARED((NPAD, d), jnp.float32),
            pltpu.VMEM_SHARED((NPAD, d), jnp.float32),
            *[pltpu.SemaphoreType.DMA for _ in range(nbuf)],
            *[pltpu.SemaphoreType.DMA for _ in range(nbuf)],
        ],
        compiler_params=_SC_PARAMS,
    )
    def k(*refs):
        y_hbms = refs[0:len(ys)]
        src_hbm = refs[len(ys)]
        dst_hbm = refs[len(ys) + 1]
        out_hbms = refs[len(ys) + 2:2 * len(ys) + 2]
        r = refs[2 * len(ys) + 2:]
        src_v, dst_v = r[0], r[1]
        rows_v = r[2:2 + nbuf]
        y_sh = r[2 + nbuf]
        acc_sh = r[3 + nbuf]
        gsem = r[4 + nbuf:4 + 2 * nbuf]
        ssem = r[4 + 2 * nbuf:4 + 3 * nbuf]
        cid = lax.axis_index("c")
        sid = lax.axis_index("s")
        wid = cid * NS + sid
        r0 = sid * RPS
        pltpu.sync_copy(src_hbm.at[wid], src_v)
        pltpu.sync_copy(dst_hbm.at[wid], dst_v)

        def gather_start(cidx, b):
            pltpu.async_copy(y_sh.at[src_v.at[cidx]], rows_v[b], gsem[b])

        def gather_wait(cidx, b):
            pltpu.make_async_copy(y_sh.at[src_v.at[cidx]], rows_v[b],
                                  gsem[b]).wait()

        def scatter_start(cidx, b):
            pltpu.async_copy(rows_v[b], acc_sh.at[dst_v.at[cidx]], ssem[b],
                             add=True)

        def scatter_wait(cidx, b):
            pltpu.make_async_copy(rows_v[b], acc_sh.at[dst_v.at[cidx]],
                                  ssem[b]).wait()

        for y_hbm, out_hbm in zip(y_hbms, out_hbms):
            pltpu.sync_copy(y_hbm.at[pl.ds(r0, RPS)],
                            acc_sh.at[pl.ds(r0, RPS)])
            pltpu.sync_copy(y_hbm.at[pl.ds(r0, RPS)],
                            y_sh.at[pl.ds(r0, RPS)])
            plsc.subcore_barrier()

            for b in range(nbuf):
                gather_start(b, b)

            @pl.loop(nbuf, nchunk, step=nbuf)
            def _(j):
                for b in range(nbuf):
                    gather_wait(j - nbuf + b, b)
                    scatter_start(j - nbuf + b, b)
                for b in range(nbuf):
                    scatter_wait(j - nbuf + b, b)
                    gather_start(j + b, b)

            for b in range(nbuf):
                gather_wait(nchunk - nbuf + b, b)
                scatter_start(nchunk - nbuf + b, b)
            for b in range(nbuf):
                scatter_wait(nchunk - nbuf + b, b)

            plsc.subcore_barrier()
            pltpu.sync_copy(acc_sh.at[pl.ds(r0, RPS)],
                            out_hbm.at[cid].at[pl.ds(r0, RPS)])

    return k(*ys, src, dst)


def _sc_degree(zeros, dst, ones):
    nchunk = dst.shape[1]
    nbuf = 4

    @functools.partial(
        pl.kernel,
        mesh=_MESH,
        out_type=jax.ShapeDtypeStruct((NC, NPAD, 16), jnp.float32),
        scratch_types=[
            pltpu.VMEM((nchunk, C), jnp.int32),
            pltpu.VMEM((C, 16), jnp.float32),
            pltpu.VMEM_SHARED((NPAD, 16), jnp.float32),
            *[pltpu.SemaphoreType.DMA for _ in range(nbuf)],
        ],
        compiler_params=_SC_PARAMS,
    )
    def k(z_hbm, dst_hbm, ones_hbm, out_hbm, dst_v, ones_v, acc_sh, *sems):
        cid = lax.axis_index("c")
        sid = lax.axis_index("s")
        wid = cid * NS + sid
        r0 = sid * RPS
        pltpu.sync_copy(dst_hbm.at[wid], dst_v)
        pltpu.sync_copy(z_hbm.at[pl.ds(r0, RPS)], acc_sh.at[pl.ds(r0, RPS)])
        pltpu.sync_copy(ones_hbm, ones_v)
        plsc.subcore_barrier()

        def scatter_start(cidx, b):
            pltpu.async_copy(ones_v, acc_sh.at[dst_v.at[cidx]], sems[b],
                             add=True)

        def scatter_wait(cidx, b):
            pltpu.make_async_copy(ones_v, acc_sh.at[dst_v.at[cidx]],
                                  sems[b]).wait()

        for b in range(nbuf):
            scatter_start(b, b)

        @pl.loop(nbuf, nchunk, step=nbuf)
        def _(j):
            for b in range(nbuf):
                scatter_wait(j - nbuf + b, b)
                scatter_start(j + b, b)

        for b in range(nbuf):
            scatter_wait(nchunk - nbuf + b, b)

        plsc.subcore_barrier()
        pltpu.sync_copy(acc_sh.at[pl.ds(r0, RPS)],
                        out_hbm.at[cid].at[pl.ds(r0, RPS)])

    return k(zeros, dst, ones)


def _blk(d):
    return pl.BlockSpec((BLK, d), lambda i: (i, 0))


def _blk3(d):
    return pl.BlockSpec((NC, BLK, d), lambda i: (0, i, 0))


def _tc_mm(x, w):
    n, kdim = x.shape
    dout = w.shape[1]

    def body(x_ref, w_ref, o_ref):
        o_ref[...] = jnp.dot(x_ref[...], w_ref[...], precision=_HI,
                             preferred_element_type=jnp.float32)

    return pl.pallas_call(
        body,
        grid=(n // BLK,),
        in_specs=[_blk(kdim), pl.BlockSpec((kdim, dout), lambda i: (0, 0))],
        out_specs=_blk(dout),
        out_shape=jax.ShapeDtypeStruct((n, dout), jnp.float32),
    )(x, w)


def _split_out(o, douts, o_refs, mask):
    off = 0
    for dk, o_ref in zip(douts, o_refs):
        o_ref[...] = jnp.where(mask, o[:, off:off + dk], 0.0)
        off += dk


def _tc_scale(xw, dp, douts):
    d = xw.shape[1]

    def body(xw_ref, dp_ref, *o_refs):
        i = pl.program_id(0)
        mask = _row_mask(i)
        deg = jnp.where(mask, dp_ref[0, :, 0:1] + dp_ref[1, :, 0:1] + 1.0,
                        1.0)
        dinv = lax.rsqrt(jnp.maximum(deg, 1e-12))
        o_refs[-1][...] = dinv
        _split_out(xw_ref[...] * dinv, douts, o_refs[:-1], mask)

    return pl.pallas_call(
        body,
        grid=(NPAD // BLK,),
        in_specs=[_blk(d), _blk3(16)],
        out_specs=[*[_blk(dk) for dk in douts], _blk(1)],
        out_shape=[*[jax.ShapeDtypeStruct((NPAD, dk), jnp.float32)
                     for dk in douts],
                   jax.ShapeDtypeStruct((NPAD, 1), jnp.float32)],
    )(xw, dp)


def _tc_layer(parts, dinv, b, w, douts):
    npart = len(parts)
    din = w.shape[0]
    dins = [y.shape[1] for _, y in parts]

    def body(*refs):
        i = pl.program_id(0)
        mask = _row_mask(i)
        zs = []
        for kk in range(npart):
            p_ref, y_ref = refs[2 * kk], refs[2 * kk + 1]
            zs.append(p_ref[0] + p_ref[1] - y_ref[...])
        z = zs[0] if npart == 1 else jnp.concatenate(zs, axis=1)
        dinv_ref, b_ref, w_ref = refs[2 * npart:2 * npart + 3]
        o_refs = refs[2 * npart + 3:]
        t = z * dinv_ref[...] + b_ref[...]
        h = jnp.where(t >= 0, t, 0.1 * t)
        o = jnp.dot(h, w_ref[...], precision=_HI,
                    preferred_element_type=jnp.float32) * dinv_ref[...]
        _split_out(o, douts, o_refs, mask)

    flat = []
    in_specs = []
    for (p, y), dk in zip(parts, dins):
        flat += [p, y]
        in_specs += [_blk3(dk), _blk(dk)]
    flat += [dinv, b, w]
    in_specs += [_blk(1), pl.BlockSpec((1, din), lambda i: (0, 0)),
                 pl.BlockSpec((din, w.shape[1]), lambda i: (0, 0))]
    return pl.pallas_call(
        body,
        grid=(NPAD // BLK,),
        in_specs=in_specs,
        out_specs=[_blk(dk) for dk in douts],
        out_shape=[jax.ShapeDtypeStruct((NPAD, dk), jnp.float32)
                   for dk in douts],
    )(*flat)


def _tc_final(p, yprev, dinv, b):
    fb = 2000

    def body(p_ref, y_ref, dinv_ref, b_ref, o_ref):
        z = p_ref[0] + p_ref[1] - y_ref[...]
        t = z[:, 0:2] * dinv_ref[...] + b_ref[...]
        m = jnp.max(t, axis=1, keepdims=True)
        e = jnp.exp(t - m)
        o_ref[...] = e / jnp.sum(e, axis=1, keepdims=True)

    return pl.pallas_call(
        body,
        grid=(N // fb,),
        in_specs=[pl.BlockSpec((NC, fb, 16), lambda i: (0, i, 0)),
                  pl.BlockSpec((fb, 16), lambda i: (i, 0)),
                  pl.BlockSpec((fb, 1), lambda i: (i, 0)),
                  pl.BlockSpec((1, 2), lambda i: (0, 0))],
        out_specs=pl.BlockSpec((fb, 2), lambda i: (i, 0)),
        out_shape=jax.ShapeDtypeStruct((N, 2), jnp.float32),
    )(p, yprev, dinv, b)


def kernel(x, edge_index, W1, b1, W2, b2, W3, b3, W4, b4):
    f32 = jnp.float32
    e = edge_index.shape[1]
    epad = -(-e // (NT * C * NBUF)) * (NT * C * NBUF)
    fill = jnp.full((epad - e,), TRASH, jnp.int32)
    src = jnp.concatenate([edge_index[0], fill]).reshape(NT, -1, C)
    dst = jnp.concatenate([edge_index[1], fill]).reshape(NT, -1, C)
    src64 = src.reshape(NT, -1, 64)
    dst64 = dst.reshape(NT, -1, 64)

    x_p = jnp.pad(x, ((0, NPAD - N), (0, 0)))
    W2p = jnp.pad(W2, ((0, 0), (0, 28)))
    b2p = jnp.pad(b2, (0, 28)).reshape(1, 128)
    W3p = jnp.pad(W3, ((0, 28), (0, 0)))
    W4p = jnp.pad(W4, ((0, 0), (0, 14)))

    zeros16 = jnp.zeros((NPAD, 16), f32)
    ones16 = jnp.ones((C, 16), f32)

    xw1 = _tc_mm(x_p, W1)
    dp = _sc_degree(zeros16, dst, ones16)
    y1a, y1b, dinv = _tc_scale(xw1, dp, (64, 64))

    pa, pb = _sc_scatter([y1a, y1b], src64, dst64)
    y2a, y2b = _tc_layer([(pa, y1a), (pb, y1b)], dinv,
                         b1.reshape(1, 128), W2p, (64, 64))
    pa, pb = _sc_scatter([y2a, y2b], src64, dst64)
    y3, = _tc_layer([(pa, y2a), (pb, y2b)], dinv, b2p, W3p, (32,))
    p, = _sc_scatter([y3], src, dst)
    y4, = _tc_layer([(p, y3)], dinv, b3.reshape(1, 32), W4p, (16,))
    p, = _sc_scatter([y4], src, dst)
    return _tc_final(p, y4, dinv, b4.reshape(1, 2))


# speedup vs baseline: 1.0448x; 1.0448x over previous
import functools

import jax
import jax.numpy as jnp
from jax import lax
from jax.experimental import pallas as pl
from jax.experimental.pallas import tpu as pltpu
from jax.experimental.pallas import tpu_sc as plsc

N = 10000
NPAD = 10240
C = 128
NC, NS = 2, 16
NT = NC * NS
RPS = NPAD // NS
BLK = 2048
TRASH = N
NBUF = 4

_HI = lax.Precision.HIGHEST
_SC_PARAMS = pltpu.CompilerParams(use_tc_tiling_on_sc=False)
_MESH = plsc.VectorSubcoreMesh(core_axis_name="c", subcore_axis_name="s")


def _row_mask(i):
    rows = lax.broadcasted_iota(jnp.int32, (BLK, 1), 0) + i * BLK
    return rows < N


def _sc_scatter(ys, src, dst):
    d = ys[0].shape[1]
    assert all(y.shape[1] == d for y in ys)
    nbuf = 2 if d >= 64 else 8
    cc = src.shape[2]
    nchunk = src.shape[1]
    assert nchunk % nbuf == 0

    @functools.partial(
        pl.kernel,
        mesh=_MESH,
        out_type=[jax.ShapeDtypeStruct((NC, NPAD, d), jnp.float32)
                  for _ in ys],
        scratch_types=[
            pltpu.VMEM((nchunk, cc), jnp.int32),
            pltpu.VMEM((nchunk, cc), jnp.int32),
            *[pltpu.VMEM((cc, d), jnp.float32) for _ in range(nbuf)],
            pltpu.VMEM_SHARED((NPAD, d), jnp.float32),
            pltpu.VMEM_SHARED((NPAD, d), jnp.float32),
            *[pltpu.SemaphoreType.DMA for _ in range(nbuf)],
            *[pltpu.SemaphoreType.DMA for _ in range(nbuf)],
        ],
        compiler_params=_SC_PARAMS,
    )
    def k(*refs):
        y_hbms = refs[0:len(ys)]
        src_hbm = refs[len(ys)]
        dst_hbm = refs[len(ys) + 1]
        out_hbms = refs[len(ys) + 2:2 * len(ys) + 2]
        r = refs[2 * len(ys) + 2:]
        src_v, dst_v = r[0], r[1]
        rows_v = r[2:2 + nbuf]
        y_sh = r[2 + nbuf]
        acc_sh = r[3 + nbuf]
        gsem = r[4 + nbuf:4 + 2 * nbuf]
        ssem = r[4 + 2 * nbuf:4 + 3 * nbuf]
        cid = lax.axis_index("c")
        sid = lax.axis_index("s")
        wid = cid * NS + sid
        r0 = sid * RPS
        pltpu.sync_copy(src_hbm.at[wid], src_v)
        pltpu.sync_copy(dst_hbm.at[wid], dst_v)

        def gather_start(cidx, b):
            pltpu.async_copy(y_sh.at[src_v.at[cidx]], rows_v[b], gsem[b])

        def gather_wait(cidx, b):
            pltpu.make_async_copy(y_sh.at[src_v.at[cidx]], rows_v[b],
                                  gsem[b]).wait()

        def scatter_start(cidx, b):
            pltpu.async_copy(rows_v[b], acc_sh.at[dst_v.at[cidx]], ssem[b],
                             add=True)

        def scatter_wait(cidx, b):
            pltpu.make_async_copy(rows_v[b], acc_sh.at[dst_v.at[cidx]],
                                  ssem[b]).wait()

        for y_hbm, out_hbm in zip(y_hbms, out_hbms):
            pltpu.sync_copy(y_hbm.at[pl.ds(r0, RPS)],
                            acc_sh.at[pl.ds(r0, RPS)])
            pltpu.sync_copy(y_hbm.at[pl.ds(r0, RPS)],
                            y_sh.at[pl.ds(r0, RPS)])
            plsc.subcore_barrier()

            for b in range(nbuf):
                gather_start(b, b)

            @pl.loop(nbuf, nchunk, step=nbuf)
            def _(j):
                for b in range(nbuf):
                    gather_wait(j - nbuf + b, b)
                    scatter_start(j - nbuf + b, b)
                for b in range(nbuf):
                    scatter_wait(j - nbuf + b, b)
                    gather_start(j + b, b)

            for b in range(nbuf):
                gather_wait(nchunk - nbuf + b, b)
                scatter_start(nchunk - nbuf + b, b)
            for b in range(nbuf):
                scatter_wait(nchunk - nbuf + b, b)

            plsc.subcore_barrier()
            pltpu.sync_copy(acc_sh.at[pl.ds(r0, RPS)],
                            out_hbm.at[cid].at[pl.ds(r0, RPS)])

    return k(*ys, src, dst)


def _sc_degree(zeros, dst, ones):
    nchunk = dst.shape[1]
    nbuf = 4

    @functools.partial(
        pl.kernel,
        mesh=_MESH,
        out_type=jax.ShapeDtypeStruct((NC, NPAD, 16), jnp.float32),
        scratch_types=[
            pltpu.VMEM((nchunk, C), jnp.int32),
            pltpu.VMEM((C, 16), jnp.float32),
            pltpu.VMEM_SHARED((NPAD, 16), jnp.float32),
            *[pltpu.SemaphoreType.DMA for _ in range(nbuf)],
        ],
        compiler_params=_SC_PARAMS,
    )
    def k(z_hbm, dst_hbm, ones_hbm, out_hbm, dst_v, ones_v, acc_sh, *sems):
        cid = lax.axis_index("c")
        sid = lax.axis_index("s")
        wid = cid * NS + sid
        r0 = sid * RPS
        pltpu.sync_copy(dst_hbm.at[wid], dst_v)
        pltpu.sync_copy(z_hbm.at[pl.ds(r0, RPS)], acc_sh.at[pl.ds(r0, RPS)])
        pltpu.sync_copy(ones_hbm, ones_v)
        plsc.subcore_barrier()

        def scatter_start(cidx, b):
            pltpu.async_copy(ones_v, acc_sh.at[dst_v.at[cidx]], sems[b],
                             add=True)

        def scatter_wait(cidx, b):
            pltpu.make_async_copy(ones_v, acc_sh.at[dst_v.at[cidx]],
                                  sems[b]).wait()

        for b in range(nbuf):
            scatter_start(b, b)

        @pl.loop(nbuf, nchunk, step=nbuf)
        def _(j):
            for b in range(nbuf):
                scatter_wait(j - nbuf + b, b)
                scatter_start(j + b, b)

        for b in range(nbuf):
            scatter_wait(nchunk - nbuf + b, b)

        plsc.subcore_barrier()
        pltpu.sync_copy(acc_sh.at[pl.ds(r0, RPS)],
                        out_hbm.at[cid].at[pl.ds(r0, RPS)])

    return k(zeros, dst, ones)


def _blk(d):
    return pl.BlockSpec((BLK, d), lambda i: (i, 0))


def _blk3(d):
    return pl.BlockSpec((NC, BLK, d), lambda i: (0, i, 0))


def _tc_mm(x, w):
    n, kdim = x.shape
    dout = w.shape[1]

    def body(x_ref, w_ref, o_ref):
        o_ref[...] = jnp.dot(x_ref[...], w_ref[...], precision=_HI,
                             preferred_element_type=jnp.float32)

    return pl.pallas_call(
        body,
        grid=(n // BLK,),
        in_specs=[_blk(kdim), pl.BlockSpec((kdim, dout), lambda i: (0, 0))],
        out_specs=_blk(dout),
        out_shape=jax.ShapeDtypeStruct((n, dout), jnp.float32),
    )(x, w)


def _split_out(o, douts, o_refs, mask):
    off = 0
    for dk, o_ref in zip(douts, o_refs):
        o_ref[...] = jnp.where(mask, o[:, off:off + dk], 0.0)
        off += dk


def _tc_scale(xw, dp, douts):
    d = xw.shape[1]

    def body(xw_ref, dp_ref, *o_refs):
        i = pl.program_id(0)
        mask = _row_mask(i)
        deg = jnp.where(mask, dp_ref[0, :, 0:1] + dp_ref[1, :, 0:1] + 1.0,
                        1.0)
        dinv = lax.rsqrt(jnp.maximum(deg, 1e-12))
        o_refs[-1][...] = dinv
        _split_out(xw_ref[...] * dinv, douts, o_refs[:-1], mask)

    return pl.pallas_call(
        body,
        grid=(NPAD // BLK,),
        in_specs=[_blk(d), _blk3(16)],
        out_specs=[*[_blk(dk) for dk in douts], _blk(1)],
        out_shape=[*[jax.ShapeDtypeStruct((NPAD, dk), jnp.float32)
                     for dk in douts],
                   jax.ShapeDtypeStruct((NPAD, 1), jnp.float32)],
    )(xw, dp)


def _tc_layer(parts, dinv, b, w, douts):
    npart = len(parts)
    din = w.shape[0]
    dins = [y.shape[1] for _, y in parts]

    def body(*refs):
        i = pl.program_id(0)
        mask = _row_mask(i)
        zs = []
        for kk in range(npart):
            p_ref, y_ref = refs[2 * kk], refs[2 * kk + 1]
            zs.append(p_ref[0] + p_ref[1] - y_ref[...])
        z = zs[0] if npart == 1 else jnp.concatenate(zs, axis=1)
        dinv_ref, b_ref, w_ref = refs[2 * npart:2 * npart + 3]
        o_refs = refs[2 * npart + 3:]
        t = z * dinv_ref[...] + b_ref[...]
        h = jnp.where(t >= 0, t, 0.1 * t)
        o = jnp.dot(h, w_ref[...], precision=_HI,
                    preferred_element_type=jnp.float32) * dinv_ref[...]
        _split_out(o, douts, o_refs, mask)

    flat = []
    in_specs = []
    for (p, y), dk in zip(parts, dins):
        flat += [p, y]
        in_specs += [_blk3(dk), _blk(dk)]
    flat += [dinv, b, w]
    in_specs += [_blk(1), pl.BlockSpec((1, din), lambda i: (0, 0)),
                 pl.BlockSpec((din, w.shape[1]), lambda i: (0, 0))]
    return pl.pallas_call(
        body,
        grid=(NPAD // BLK,),
        in_specs=in_specs,
        out_specs=[_blk(dk) for dk in douts],
        out_shape=[jax.ShapeDtypeStruct((NPAD, dk), jnp.float32)
                   for dk in douts],
    )(*flat)


def _tc_final(p, yprev, dinv, b):
    fb = 2000

    def body(p_ref, y_ref, dinv_ref, b_ref, o_ref):
        z = p_ref[0] + p_ref[1] - y_ref[...]
        t = z[:, 0:2] * dinv_ref[...] + b_ref[...]
        m = jnp.max(t, axis=1, keepdims=True)
        e = jnp.exp(t - m)
        o_ref[...] = e / jnp.sum(e, axis=1, keepdims=True)

    return pl.pallas_call(
        body,
        grid=(N // fb,),
        in_specs=[pl.BlockSpec((NC, fb, 16), lambda i: (0, i, 0)),
                  pl.BlockSpec((fb, 16), lambda i: (i, 0)),
                  pl.BlockSpec((fb, 1), lambda i: (i, 0)),
                  pl.BlockSpec((1, 2), lambda i: (0, 0))],
        out_specs=pl.BlockSpec((fb, 2), lambda i: (i, 0)),
        out_shape=jax.ShapeDtypeStruct((N, 2), jnp.float32),
    )(p, yprev, dinv, b)


def kernel(x, edge_index, W1, b1, W2, b2, W3, b3, W4, b4):
    f32 = jnp.float32
    e = edge_index.shape[1]
    epad = -(-e // (NT * C * NBUF)) * (NT * C * NBUF)
    fill = jnp.full((epad - e,), TRASH, jnp.int32)
    src = jnp.concatenate([edge_index[0], fill]).reshape(NT, -1, C)
    dst = jnp.concatenate([edge_index[1], fill]).reshape(NT, -1, C)

    x_p = jnp.pad(x, ((0, NPAD - N), (0, 0)))
    W2p = jnp.pad(W2, ((0, 0), (0, 28)))
    b2p = jnp.pad(b2, (0, 28)).reshape(1, 128)
    W3p = jnp.pad(W3, ((0, 28), (0, 0)))
    W4p = jnp.pad(W4, ((0, 0), (0, 14)))

    zeros16 = jnp.zeros((NPAD, 16), f32)
    ones16 = jnp.ones((C, 16), f32)

    xw1 = _tc_mm(x_p, W1)
    dp = _sc_degree(zeros16, dst, ones16)
    y1a, y1b, dinv = _tc_scale(xw1, dp, (64, 64))

    pa, pb = _sc_scatter([y1a, y1b], src, dst)
    y2a, y2b = _tc_layer([(pa, y1a), (pb, y1b)], dinv,
                         b1.reshape(1, 128), W2p, (64, 64))
    pa, pb = _sc_scatter([y2a, y2b], src, dst)
    y3, = _tc_layer([(pa, y2a), (pb, y2b)], dinv, b2p, W3p, (32,))
    p, = _sc_scatter([y3], src, dst)
    y4, = _tc_layer([(p, y3)], dinv, b3.reshape(1, 32), W4p, (16,))
    p, = _sc_scatter([y4], src, dst)
    return _tc_final(p, y4, dinv, b4.reshape(1, 2))
